# Initial kernel scaffold; baseline (speedup 1.0000x reference)
import jax
import jax.numpy as jnp
from jax.experimental import pallas as pl


def kernel(x, W, b):
    raise NotImplementedError("write your pallas kernel here")


# speedup vs baseline: 2.5338x; 2.5338x over previous
import functools

import jax
import jax.numpy as jnp
from jax.experimental import pallas as pl
from jax.experimental.pallas import tpu as pltpu

_E = 16
_ROWS = 8192
_R = 512
_NEG = -3.0e38


def _fused_body(x_ref, w_ref, b_ref, gates_ref, idx_ref, loss_ref,
                imp_acc, load_acc):
    i = pl.program_id(0)
    nsteps = pl.num_programs(0)

    logits = jnp.dot(x_ref[...], w_ref[...],
                     preferred_element_type=jnp.float32) + b_ref[...]

    cols = jax.lax.broadcasted_iota(jnp.int32, logits.shape, 1)
    m1 = jnp.max(logits, axis=1, keepdims=True)
    i1 = jnp.min(jnp.where(logits == m1, cols, _E), axis=1, keepdims=True)
    masked = jnp.where(cols == i1, _NEG, logits)
    m2 = jnp.max(masked, axis=1, keepdims=True)
    i2 = jnp.min(jnp.where(masked == m2, cols, _E), axis=1, keepdims=True)

    e = jnp.exp(m2 - m1)
    denom = 1.0 + e
    g1 = 1.0 / denom
    g2 = e / denom

    gates = jnp.where(cols == i1, g1, jnp.where(cols == i2, g2, 0.0))
    gates_ref[...] = gates
    idx_ref[...] = jnp.concatenate([i1, i2], axis=1)

    p = jnp.exp(logits - m1)
    load_rows = p / jnp.sum(p, axis=1, keepdims=True)

    @pl.when(i == 0)
    def _init():
        imp_acc[...] = jnp.zeros_like(imp_acc)
        load_acc[...] = jnp.zeros_like(load_acc)

    imp_acc[...] += jnp.sum(gates, axis=0, keepdims=True)
    load_acc[...] += jnp.sum(load_rows, axis=0, keepdims=True)

    @pl.when(i == nsteps - 1)
    def _finish():
        def cv(v):
            mean = jnp.sum(v) / _E
            var = jnp.sum((v - mean) ** 2) / (_E - 1)
            return jnp.sqrt(var) / (mean + 1e-6)
        loss_ref[...] = jnp.reshape(cv(imp_acc[...]) + cv(load_acc[...]), (1, 1))


@functools.partial(jax.jit, static_argnames=())
def kernel(x, W, b):
    nsteps = _ROWS // _R
    gates, idx, loss = pl.pallas_call(
        _fused_body,
        grid=(nsteps,),
        in_specs=[
            pl.BlockSpec((_R, 2048), lambda i: (i, 0)),
            pl.BlockSpec((2048, _E), lambda i: (0, 0)),
            pl.BlockSpec((1, _E), lambda i: (0, 0)),
        ],
        out_specs=[
            pl.BlockSpec((_R, _E), lambda i: (i, 0)),
            pl.BlockSpec((_R, 2), lambda i: (i, 0)),
            pl.BlockSpec((1, 1), lambda i: (0, 0)),
        ],
        out_shape=[
            jax.ShapeDtypeStruct((_ROWS, _E), jnp.float32),
            jax.ShapeDtypeStruct((_ROWS, 2), jnp.int32),
            jax.ShapeDtypeStruct((1, 1), jnp.float32),
        ],
        scratch_shapes=[
            pltpu.VMEM((1, _E), jnp.float32),
            pltpu.VMEM((1, _E), jnp.float32),
        ],
    )(x, W, b.reshape(1, _E))
    return gates, idx, jnp.reshape(loss, ())
